# baseline (speedup 1.0000x reference)
import jax
import jax.numpy as jnp
from jax.experimental import pallas as pl


def _router_block(x_ref, w_ref, noise_ref, out_ref):
    out_ref[...] = jax.lax.dot_general(
        x_ref[...],
        w_ref[...],
        dimension_numbers=(((1,), (1,)), ((), ())),
        preferred_element_type=jnp.float32,
    ) + noise_ref[...]


def kernel(x, W, noise):
    tokens, d_model = x.shape
    n_experts = W.shape[0]
    blk = 512
    return pl.pallas_call(
        _router_block,
        grid=(tokens // blk,),
        in_specs=[
            pl.BlockSpec((blk, d_model), lambda i: (i, 0)),
            pl.BlockSpec((n_experts, d_model), lambda i: (0, 0)),
            pl.BlockSpec((blk, n_experts), lambda i: (i, 0)),
        ],
        out_specs=pl.BlockSpec((blk, n_experts), lambda i: (i, 0)),
        out_shape=jax.ShapeDtypeStruct((tokens, n_experts), jnp.float32),
    )(x, W, noise)


# speedup vs baseline: 1.0038x; 1.0038x over previous
import jax
import jax.numpy as jnp
from jax.experimental import pallas as pl


def _router_block(x_ref, w_ref, noise_ref, out_ref):
    out_ref[...] = jax.lax.dot_general(
        x_ref[...].astype(jnp.bfloat16),
        w_ref[...].astype(jnp.bfloat16),
        dimension_numbers=(((1,), (1,)), ((), ())),
        preferred_element_type=jnp.float32,
    ) + noise_ref[...]


def kernel(x, W, noise):
    tokens, d_model = x.shape
    n_experts = W.shape[0]
    blk = 512
    return pl.pallas_call(
        _router_block,
        grid=(tokens // blk,),
        in_specs=[
            pl.BlockSpec((blk, d_model), lambda i: (i, 0)),
            pl.BlockSpec((n_experts, d_model), lambda i: (0, 0)),
            pl.BlockSpec((blk, n_experts), lambda i: (i, 0)),
        ],
        out_specs=pl.BlockSpec((blk, n_experts), lambda i: (i, 0)),
        out_shape=jax.ShapeDtypeStruct((tokens, n_experts), jnp.float32),
    )(x, W, noise)
